# Initial kernel scaffold; baseline (speedup 1.0000x reference)
import jax
import jax.numpy as jnp
from jax.experimental import pallas as pl


def kernel(feats, edge_index, enc1_W, enc1_b, enc2_W, enc2_b, theta0_W, theta0_b, phi0_W, phi0_b, theta1_W, theta1_b, phi1_W, phi1_b, theta2_W, theta2_b, phi2_W, phi2_b, dec1_W, dec1_b, dec2_W, dec2_b):
    raise NotImplementedError("write your pallas kernel here")


# speedup vs baseline: 1.3625x; 1.3625x over previous
import functools

import jax
import jax.numpy as jnp
from jax import lax
from jax.experimental import pallas as pl
from jax.experimental.pallas import tpu as pltpu
from jax.experimental.pallas import tpu_sc as plsc

N = 10000
E = 320000
H = 128
THRESHOLD = 0.01

NUM_WORKERS = 32
NODES_PER_TILE = 320
NP = NUM_WORKERS * NODES_PER_TILE

CHUNK = 8192
GROUPS = CHUNK // 16
GB = 64
ROW_BLK = 512

_mesh = plsc.VectorSubcoreMesh(
    core_axis_name="c", subcore_axis_name="s", num_cores=2, num_subcores=16)


@functools.partial(
    pl.kernel,
    out_type=jax.ShapeDtypeStruct((NP, H), jnp.float32),
    mesh=_mesh,
    compiler_params=pltpu.CompilerParams(needs_layout_passes=False),
    scratch_types=[
        pltpu.VMEM((NODES_PER_TILE, H), jnp.float32),
        pltpu.VMEM((CHUNK,), jnp.int32),
        pltpu.VMEM((CHUNK,), jnp.int32),
        pltpu.VMEM((CHUNK,), jnp.int32),
        pltpu.VMEM((CHUNK + 16,), jnp.int32),
        pltpu.VMEM((GB, H), jnp.float32),
        pltpu.SemaphoreType.DMA,
    ],
)
def _segmax(src_hbm, dst_hbm, a_hbm, s_hbm,
            acc, dstv, srcv, gsrc, gdst, rows, sem):
    wid = lax.axis_index("s") * 2 + lax.axis_index("c")
    lo = wid * NODES_PER_TILE

    neg = jnp.full((16,), -jnp.inf, jnp.float32)
    zero16 = jnp.zeros((16,), jnp.int32)

    def init_body(r, carry):
        for j in range(8):
            acc[r, pl.ds(j * 16, 16)] = neg
        return carry
    lax.fori_loop(0, NODES_PER_TILE, init_body, 0)

    def chunk_body(k, carry):
        pltpu.sync_copy(dst_hbm.at[pl.ds(k * CHUNK, CHUNK)], dstv)
        pltpu.sync_copy(src_hbm.at[pl.ds(k * CHUNK, CHUNK)], srcv)

        def z_body(g, carry):
            gsrc[pl.ds(g * 16, 16)] = zero16
            return carry
        lax.fori_loop(0, GROUPS, z_body, 0)

        def f_body(g, cnt):
            d = dstv[pl.ds(g * 16, 16)]
            s = srcv[pl.ds(g * 16, 16)]
            m = (d >= lo) & (d < lo + NODES_PER_TILE)
            c = plsc.cumsum(jnp.where(m, jnp.int32(1), jnp.int32(0)))
            pos = (cnt - 1) + c
            plsc.store_scatter(gsrc, [pos], s, mask=m)
            plsc.store_scatter(gdst, [pos], d - lo, mask=m)
            return cnt + jnp.max(c)
        cnt = lax.fori_loop(0, GROUPS, f_body, jnp.int32(0))

        nblk = (cnt + (GB - 1)) // GB

        def blk_body(b, carry):
            pltpu.async_copy(a_hbm.at[gsrc.at[pl.ds(b * GB, GB)]], rows,
                             sem).wait()
            nedge = jnp.minimum(GB, cnt - b * GB)

            def e_body(i, carry):
                dl = gdst[pl.ds(b * GB + i, 16)][0]
                for j in range(8):
                    sl = pl.ds(j * 16, 16)
                    acc[dl, sl] = jnp.maximum(acc[dl, sl], rows[i, sl])
                return carry
            lax.fori_loop(0, nedge, e_body, 0)
            return carry
        lax.fori_loop(0, nblk, blk_body, 0)
        return carry
    lax.fori_loop(0, E // CHUNK, chunk_body, 0)

    pltpu.sync_copy(acc, s_hbm.at[pl.ds(lo, NODES_PER_TILE)])


def _dot(a, b):
    return jnp.dot(a, b, preferred_element_type=jnp.float32)


def _enc_body(f_ref, w1_ref, b1_ref, w2_ref, b2_ref, tw_ref, pw_ref,
              tb_ref, pb_ref, a_ref, bout_ref):
    x = jnp.maximum(_dot(f_ref[...], w1_ref[...]) + b1_ref[...], 0.0)
    x = jnp.maximum(_dot(x, w2_ref[...]) + b2_ref[...], 0.0)
    a_ref[...] = _dot(x, tw_ref[...])
    bout_ref[...] = (_dot(x, pw_ref[...] - tw_ref[...])
                     + (tb_ref[...] + pb_ref[...]))


def _layer_body(s_ref, bmat_ref, tw_ref, pw_ref, tb_ref, pb_ref,
                a_ref, bout_ref):
    x = jnp.maximum(bmat_ref[...] + s_ref[...], 0.0)
    a_ref[...] = _dot(x, tw_ref[...])
    bout_ref[...] = (_dot(x, pw_ref[...] - tw_ref[...])
                     + (tb_ref[...] + pb_ref[...]))


def _dec_body(s_ref, bmat_ref, w1_ref, b1_ref, w2_ref, b2_ref, out_ref):
    x = jnp.maximum(bmat_ref[...] + s_ref[...], 0.0)
    h = jnp.maximum(_dot(x, w1_ref[...]) + b1_ref[...], 0.0)
    o = _dot(h, w2_ref[...]) + b2_ref[...]
    out_ref[...] = jnp.where(o < THRESHOLD, 0.0, o)


_row_spec = pl.BlockSpec((ROW_BLK, H), lambda i: (i, 0))
_w_spec = pl.BlockSpec((H, H), lambda i: (0, 0))
_b_spec = pl.BlockSpec((1, H), lambda i: (0, 0))
_out2 = (jax.ShapeDtypeStruct((NP, H), jnp.float32),
         jax.ShapeDtypeStruct((NP, H), jnp.float32))

_enc_call = pl.pallas_call(
    _enc_body,
    grid=(NP // ROW_BLK,),
    in_specs=[_row_spec, _w_spec, _b_spec, _w_spec, _b_spec,
              _w_spec, _w_spec, _b_spec, _b_spec],
    out_specs=(_row_spec, _row_spec),
    out_shape=_out2,
)

_layer_call = pl.pallas_call(
    _layer_body,
    grid=(NP // ROW_BLK,),
    in_specs=[_row_spec, _row_spec, _w_spec, _w_spec, _b_spec, _b_spec],
    out_specs=(_row_spec, _row_spec),
    out_shape=_out2,
)

_dec_call = pl.pallas_call(
    _dec_body,
    grid=(NP // ROW_BLK,),
    in_specs=[_row_spec, _row_spec, _w_spec, _b_spec, _w_spec, _b_spec],
    out_specs=_row_spec,
    out_shape=jax.ShapeDtypeStruct((NP, H), jnp.float32),
)


def kernel(feats, edge_index, enc1_W, enc1_b, enc2_W, enc2_b,
           theta0_W, theta0_b, phi0_W, phi0_b,
           theta1_W, theta1_b, phi1_W, phi1_b,
           theta2_W, theta2_b, phi2_W, phi2_b,
           dec1_W, dec1_b, dec2_W, dec2_b):
    src = edge_index[0]
    dst = edge_index[1]
    featsp = jnp.pad(feats, ((0, NP - N), (0, 0)))
    r = lambda v: v.reshape(1, H)

    A, B = _enc_call(featsp, enc1_W, r(enc1_b), enc2_W, r(enc2_b),
                     theta0_W, phi0_W, r(theta0_b), r(phi0_b))
    S = _segmax(src, dst, A)
    A, B = _layer_call(S, B, theta1_W, phi1_W, r(theta1_b), r(phi1_b))
    S = _segmax(src, dst, A)
    A, B = _layer_call(S, B, theta2_W, phi2_W, r(theta2_b), r(phi2_b))
    S = _segmax(src, dst, A)

    dec2_Wp = jnp.pad(dec2_W, ((0, 0), (0, H - dec2_W.shape[1])))
    dec2_bp = jnp.pad(dec2_b, (0, H - dec2_b.shape[0]))
    out = _dec_call(S, B, dec1_W, r(dec1_b), dec2_Wp, dec2_bp.reshape(1, H))
    return out[:N, :1]


# speedup vs baseline: 1.6919x; 1.2418x over previous
import functools

import jax
import jax.numpy as jnp
from jax import lax
from jax.experimental import pallas as pl
from jax.experimental.pallas import tpu as pltpu
from jax.experimental.pallas import tpu_sc as plsc

N = 10000
E = 320000
H = 128
THRESHOLD = 0.01

NUM_WORKERS = 32
NODES_PER_TILE = 320
NP = NUM_WORKERS * NODES_PER_TILE

CHUNK = 8192
GROUPS = CHUNK // 16
LISTCAP = E + CHUNK
LC = 2048
GB = 256
NB = LC // GB
ROW_BLK = 512

PACK_SHIFT = 14
PACK_MASK = (1 << PACK_SHIFT) - 1

_mesh = plsc.VectorSubcoreMesh(
    core_axis_name="c", subcore_axis_name="s", num_cores=2, num_subcores=16)
_sc_params = pltpu.CompilerParams(needs_layout_passes=False)


def _wid():
    return lax.axis_index("s") * 2 + lax.axis_index("c")


@functools.partial(
    pl.kernel,
    out_type=(jax.ShapeDtypeStruct((NUM_WORKERS * LISTCAP,), jnp.int32),
              jax.ShapeDtypeStruct((NUM_WORKERS * 16,), jnp.int32)),
    mesh=_mesh,
    compiler_params=_sc_params,
    scratch_types=[
        pltpu.VMEM((CHUNK,), jnp.int32),
        pltpu.VMEM((CHUNK,), jnp.int32),
        pltpu.VMEM((2 * CHUNK,), jnp.int32),
        pltpu.VMEM((16,), jnp.int32),
    ],
)
def _build_lists(src_hbm, dst_hbm, lists_hbm, counts_hbm,
                 dstv, srcv, buf, cntv):
    wid = _wid()
    lo = wid * NODES_PER_TILE

    def chunk_body(k, carry):
        cnt_v, fl = carry
        pltpu.sync_copy(dst_hbm.at[pl.ds(pl.multiple_of(k * CHUNK, 8), CHUNK)], dstv)
        pltpu.sync_copy(src_hbm.at[pl.ds(pl.multiple_of(k * CHUNK, 8), CHUNK)], srcv)

        def f_body(g, cnt_v):
            d = dstv[pl.ds(g * 16, 16)]
            s = srcv[pl.ds(g * 16, 16)]
            m = (d >= lo) & (d < lo + NODES_PER_TILE)
            c = plsc.cumsum(jnp.where(m, jnp.int32(1), jnp.int32(0)))
            pos = (cnt_v - 1) + c
            packed = ((d - lo) << PACK_SHIFT) | s
            plsc.store_scatter(buf, [pos], packed, mask=m)
            return cnt_v + plsc.all_reduce_population_count(m)
        cnt_v = lax.fori_loop(0, GROUPS, f_body, cnt_v)

        cb = cnt_v[0]

        @pl.when(cb >= CHUNK)
        def _flush():
            pltpu.sync_copy(buf.at[pl.ds(0, CHUNK)],
                            lists_hbm.at[pl.ds(pl.multiple_of(wid * LISTCAP + fl, 8), CHUNK)])
            nmv = (cb - CHUNK + 15) // 16

            def mv(k2, carry):
                v = buf[pl.ds(CHUNK + k2 * 16, 16)]
                buf[pl.ds(k2 * 16, 16)] = v
                return carry
            lax.fori_loop(0, nmv, mv, 0)

        did = jnp.where(cb >= CHUNK, jnp.int32(CHUNK), jnp.int32(0))
        return cnt_v - did, fl + did

    cnt_v0 = jnp.zeros((16,), jnp.int32)
    cnt_v, fl = lax.fori_loop(0, E // CHUNK, chunk_body,
                              (cnt_v0, jnp.int32(0)))
    pltpu.sync_copy(buf.at[pl.ds(0, CHUNK)],
                    lists_hbm.at[pl.ds(pl.multiple_of(wid * LISTCAP + fl, 8), CHUNK)])
    cntv[pl.ds(0, 16)] = cnt_v + fl
    pltpu.sync_copy(cntv, counts_hbm.at[pl.ds(pl.multiple_of(wid * 16, 8), 16)])


@functools.partial(
    pl.kernel,
    out_type=jax.ShapeDtypeStruct((NP, H), jnp.float32),
    mesh=_mesh,
    compiler_params=_sc_params,
    scratch_types=[
        pltpu.VMEM((NODES_PER_TILE, H), jnp.float32),
        pltpu.VMEM((LC,), jnp.int32),
        pltpu.VMEM((LC,), jnp.int32),
        pltpu.VMEM((LC,), jnp.int32),
        pltpu.VMEM((GB, H), jnp.float32),
        pltpu.VMEM((GB, H), jnp.float32),
        pltpu.VMEM((16,), jnp.int32),
        pltpu.SemaphoreType.DMA,
        pltpu.SemaphoreType.DMA,
    ],
)
def _segmax(lists_hbm, counts_hbm, a_hbm, s_hbm,
            acc, lbuf, gsrc, gdl, rows0, rows1, cntv, sem0, sem1):
    wid = _wid()
    lo = wid * NODES_PER_TILE

    neg = jnp.full((16,), -jnp.inf, jnp.float32)

    def init_body(r, carry):
        for j in range(8):
            acc[r, pl.ds(j * 16, 16)] = neg
        return carry
    lax.fori_loop(0, NODES_PER_TILE, init_body, 0)

    pltpu.sync_copy(counts_hbm.at[pl.ds(pl.multiple_of(wid * 16, 8), 16)], cntv)
    cnt = cntv[pl.ds(0, 16)][0]
    iota = lax.iota(jnp.int32, 16)
    cnt_splat = cnt + jnp.zeros((16,), jnp.int32)
    iotas = [iota + 16 * j for j in range(8)]
    rows = (rows0, rows1)
    sems = (sem0, sem1)

    def lc_body(lc, carry):
        base_lc = lc * LC
        pltpu.sync_copy(lists_hbm.at[pl.ds(pl.multiple_of(wid * LISTCAP + base_lc, 8), LC)], lbuf)

        def unpack_body(g, carry):
            p = lbuf[pl.ds(g * 16, 16)]
            valid = (base_lc + g * 16 + iota) < cnt_splat
            gsrc[pl.ds(g * 16, 16)] = jnp.where(valid, p & PACK_MASK, 0)
            gdl[pl.ds(g * 16, 16)] = lax.shift_right_logical(p, PACK_SHIFT)
            return carry
        lax.fori_loop(0, LC // 16, unpack_body, 0)

        cps = [None] * NB
        cps[0] = pltpu.async_copy(a_hbm.at[gsrc.at[pl.ds(0, GB)]],
                                  rows[0], sems[0])
        for b in range(NB):
            if b + 1 < NB:
                cps[b + 1] = pltpu.async_copy(
                    a_hbm.at[gsrc.at[pl.ds((b + 1) * GB, GB)]],
                    rows[(b + 1) % 2], sems[(b + 1) % 2])
            cps[b].wait()
            rcur = rows[b % 2]
            nb_e = jnp.clip(cnt - (base_lc + b * GB), 0, GB)

            def acc_g(g2, carry):
                dlv = gdl[pl.ds(b * GB + g2 * 16, 16)]
                ne_in = jnp.minimum(16, nb_e - g2 * 16)

                def acc_e(i2, carry):
                    dlb = dlv.at[jnp.full((16,), i2, jnp.int32)].get(
                        mode="promise_in_bounds")
                    r = g2 * 16 + i2
                    for j in range(8):
                        ag = plsc.load_gather(acc, [dlb, iotas[j]])
                        rv = rcur[r, pl.ds(16 * j, 16)]
                        plsc.store_scatter(acc, [dlb, iotas[j]],
                                           jnp.maximum(ag, rv))
                    return carry
                lax.fori_loop(0, ne_in, acc_e, 0)
                return carry
            lax.fori_loop(0, (nb_e + 15) // 16, acc_g, 0)
        return carry

    nlc = (cnt + (LC - 1)) // LC
    lax.fori_loop(0, nlc, lc_body, 0)

    pltpu.sync_copy(acc, s_hbm.at[pl.ds(lo, NODES_PER_TILE)])


def _dot(a, b):
    return jnp.dot(a, b, preferred_element_type=jnp.float32)


def _enc_body(f_ref, w1_ref, b1_ref, w2_ref, b2_ref, tw_ref, pw_ref,
              tb_ref, pb_ref, a_ref, bout_ref):
    x = jnp.maximum(_dot(f_ref[...], w1_ref[...]) + b1_ref[...], 0.0)
    x = jnp.maximum(_dot(x, w2_ref[...]) + b2_ref[...], 0.0)
    a_ref[...] = _dot(x, tw_ref[...])
    bout_ref[...] = (_dot(x, pw_ref[...] - tw_ref[...])
                     + (tb_ref[...] + pb_ref[...]))


def _layer_body(s_ref, bmat_ref, tw_ref, pw_ref, tb_ref, pb_ref,
                a_ref, bout_ref):
    x = jnp.maximum(bmat_ref[...] + s_ref[...], 0.0)
    a_ref[...] = _dot(x, tw_ref[...])
    bout_ref[...] = (_dot(x, pw_ref[...] - tw_ref[...])
                     + (tb_ref[...] + pb_ref[...]))


def _dec_body(s_ref, bmat_ref, w1_ref, b1_ref, w2_ref, b2_ref, out_ref):
    x = jnp.maximum(bmat_ref[...] + s_ref[...], 0.0)
    h = jnp.maximum(_dot(x, w1_ref[...]) + b1_ref[...], 0.0)
    o = _dot(h, w2_ref[...]) + b2_ref[...]
    out_ref[...] = jnp.where(o < THRESHOLD, 0.0, o)


_row_spec = pl.BlockSpec((ROW_BLK, H), lambda i: (i, 0))
_w_spec = pl.BlockSpec((H, H), lambda i: (0, 0))
_b_spec = pl.BlockSpec((1, H), lambda i: (0, 0))
_out2 = (jax.ShapeDtypeStruct((NP, H), jnp.float32),
         jax.ShapeDtypeStruct((NP, H), jnp.float32))

_enc_call = pl.pallas_call(
    _enc_body,
    grid=(NP // ROW_BLK,),
    in_specs=[_row_spec, _w_spec, _b_spec, _w_spec, _b_spec,
              _w_spec, _w_spec, _b_spec, _b_spec],
    out_specs=(_row_spec, _row_spec),
    out_shape=_out2,
)

_layer_call = pl.pallas_call(
    _layer_body,
    grid=(NP // ROW_BLK,),
    in_specs=[_row_spec, _row_spec, _w_spec, _w_spec, _b_spec, _b_spec],
    out_specs=(_row_spec, _row_spec),
    out_shape=_out2,
)

_dec_call = pl.pallas_call(
    _dec_body,
    grid=(NP // ROW_BLK,),
    in_specs=[_row_spec, _row_spec, _w_spec, _b_spec, _w_spec, _b_spec],
    out_specs=_row_spec,
    out_shape=jax.ShapeDtypeStruct((NP, H), jnp.float32),
)


def kernel(feats, edge_index, enc1_W, enc1_b, enc2_W, enc2_b,
           theta0_W, theta0_b, phi0_W, phi0_b,
           theta1_W, theta1_b, phi1_W, phi1_b,
           theta2_W, theta2_b, phi2_W, phi2_b,
           dec1_W, dec1_b, dec2_W, dec2_b):
    src = edge_index[0]
    dst = edge_index[1]
    featsp = jnp.pad(feats, ((0, NP - N), (0, 0)))
    r = lambda v: v.reshape(1, H)

    lists, counts = _build_lists(src, dst)

    A, B = _enc_call(featsp, enc1_W, r(enc1_b), enc2_W, r(enc2_b),
                     theta0_W, phi0_W, r(theta0_b), r(phi0_b))
    S = _segmax(lists, counts, A)
    A, B = _layer_call(S, B, theta1_W, phi1_W, r(theta1_b), r(phi1_b))
    S = _segmax(lists, counts, A)
    A, B = _layer_call(S, B, theta2_W, phi2_W, r(theta2_b), r(phi2_b))
    S = _segmax(lists, counts, A)

    dec2_Wp = jnp.pad(dec2_W, ((0, 0), (0, H - dec2_W.shape[1])))
    dec2_bp = jnp.pad(dec2_b, (0, H - dec2_b.shape[0]))
    out = _dec_call(S, B, dec1_W, r(dec1_b), dec2_Wp, dec2_bp.reshape(1, H))
    return out[:N, :1]


# speedup vs baseline: 3.7809x; 2.2347x over previous
import functools

import jax
import jax.numpy as jnp
from jax import lax
from jax.experimental import pallas as pl
from jax.experimental.pallas import tpu as pltpu
from jax.experimental.pallas import tpu_sc as plsc

N = 10000
E = 320000
H = 128
THRESHOLD = 0.01

NUM_WORKERS = 32
NODES_PER_TILE = 320
NP = NUM_WORKERS * NODES_PER_TILE

CHUNK = 8192
GROUPS = CHUNK // 16
LISTCAP = E + CHUNK
GB = 256
ROW_BLK = 512

PACK_SHIFT = 14
PACK_MASK = (1 << PACK_SHIFT) - 1

_mesh = plsc.VectorSubcoreMesh(
    core_axis_name="c", subcore_axis_name="s", num_cores=2, num_subcores=16)
_sc_params = pltpu.CompilerParams(needs_layout_passes=False)


def _wid():
    return lax.axis_index("s") * 2 + lax.axis_index("c")


@functools.partial(
    pl.kernel,
    out_type=(jax.ShapeDtypeStruct((NUM_WORKERS * LISTCAP,), jnp.int32),
              jax.ShapeDtypeStruct((NUM_WORKERS * 16,), jnp.int32)),
    mesh=_mesh,
    compiler_params=_sc_params,
    scratch_types=[
        pltpu.VMEM((CHUNK,), jnp.int32),
        pltpu.VMEM((CHUNK,), jnp.int32),
        pltpu.VMEM((2 * CHUNK,), jnp.int32),
        pltpu.VMEM((16,), jnp.int32),
    ],
)
def _build_lists(src_hbm, dst_hbm, lists_hbm, counts_hbm,
                 dstv, srcv, buf, cntv):
    wid = _wid()
    lo = wid * NODES_PER_TILE

    def chunk_body(k, carry):
        cnt_v, fl = carry
        pltpu.sync_copy(dst_hbm.at[pl.ds(pl.multiple_of(k * CHUNK, 8), CHUNK)], dstv)
        pltpu.sync_copy(src_hbm.at[pl.ds(pl.multiple_of(k * CHUNK, 8), CHUNK)], srcv)

        def f_body(g, cnt_v):
            d = dstv[pl.ds(g * 16, 16)]
            s = srcv[pl.ds(g * 16, 16)]
            m = (d >= lo) & (d < lo + NODES_PER_TILE)
            c = plsc.cumsum(jnp.where(m, jnp.int32(1), jnp.int32(0)))
            pos = (cnt_v - 1) + c
            packed = ((d - lo) << PACK_SHIFT) | s
            plsc.store_scatter(buf, [pos], packed, mask=m)
            return cnt_v + plsc.all_reduce_population_count(m)
        cnt_v = lax.fori_loop(0, GROUPS, f_body, cnt_v)

        cb = cnt_v[0]

        @pl.when(cb >= CHUNK)
        def _flush():
            pltpu.sync_copy(buf.at[pl.ds(0, CHUNK)],
                            lists_hbm.at[pl.ds(pl.multiple_of(wid * LISTCAP + fl, 8), CHUNK)])
            nmv = (cb - CHUNK + 15) // 16

            def mv(k2, carry):
                v = buf[pl.ds(CHUNK + k2 * 16, 16)]
                buf[pl.ds(k2 * 16, 16)] = v
                return carry
            lax.fori_loop(0, nmv, mv, 0)

        did = jnp.where(cb >= CHUNK, jnp.int32(CHUNK), jnp.int32(0))
        return cnt_v - did, fl + did

    cnt_v0 = jnp.zeros((16,), jnp.int32)
    cnt_v, fl = lax.fori_loop(0, E // CHUNK, chunk_body,
                              (cnt_v0, jnp.int32(0)))
    pltpu.sync_copy(buf.at[pl.ds(0, CHUNK)],
                    lists_hbm.at[pl.ds(pl.multiple_of(wid * LISTCAP + fl, 8), CHUNK)])
    cntv[pl.ds(0, 16)] = cnt_v + fl
    pltpu.sync_copy(cntv, counts_hbm.at[pl.ds(pl.multiple_of(wid * 16, 8), 16)])


SCK = 8192
DUMP = NODES_PER_TILE


@functools.partial(
    pl.kernel,
    out_type=jax.ShapeDtypeStruct((NP, H), jnp.float32),
    mesh=_mesh,
    compiler_params=_sc_params,
    scratch_types=[
        pltpu.VMEM((NODES_PER_TILE + 8, H), jnp.float32),
        pltpu.VMEM((SCK,), jnp.int32),
        pltpu.VMEM((SCK,), jnp.int32),
        pltpu.VMEM((GB, H), jnp.float32),
        pltpu.VMEM((GB, H), jnp.float32),
        pltpu.VMEM((16,), jnp.int32),
        pltpu.SemaphoreType.DMA,
        pltpu.SemaphoreType.DMA,
    ],
)
def _segmax(lists_hbm, counts_hbm, a_hbm, s_hbm,
            acc, lbuf, gdl, rows0, rows1, cntv, sem0, sem1):
    wid = _wid()
    lo = wid * NODES_PER_TILE

    neg = jnp.full((16,), -jnp.inf, jnp.float32)

    def init_body(r, carry):
        for j in range(8):
            acc[r, pl.ds(j * 16, 16)] = neg
        return carry
    lax.fori_loop(0, NODES_PER_TILE, init_body, 0)

    pltpu.sync_copy(counts_hbm.at[pl.ds(pl.multiple_of(wid * 16, 8), 16)],
                    cntv)
    cnt = cntv[pl.ds(0, 16)][0]
    iota = lax.iota(jnp.int32, 16)
    cnt_splat = cnt + jnp.zeros((16,), jnp.int32)
    iotas = [iota + 16 * j for j in range(8)]

    def gather_blk(b, rbuf, sem):
        return pltpu.async_copy(
            a_hbm.at[lbuf.at[pl.ds(pl.multiple_of(b * GB, 8), GB)]],
            rbuf, sem)

    def process_blk(b, rbuf):
        def grp(g2, carry):
            off = pl.multiple_of(b * GB + g2 * 16, 8)
            dlv = gdl[pl.ds(off, 16)]
            for i2 in range(16):
                dlb = dlv.at[jnp.full((16,), i2, jnp.int32)].get(
                    mode="promise_in_bounds")
                for j in range(8):
                    ag = plsc.load_gather(acc, [dlb, iotas[j]])
                    rv = rbuf[g2 * 16 + i2, pl.ds(16 * j, 16)]
                    plsc.store_scatter(acc, [dlb, iotas[j]],
                                       jnp.maximum(ag, rv))
            return carry
        lax.fori_loop(0, GB // 16, grp, 0)

    def sc_body(scix, carry):
        base_sc = scix * SCK
        pltpu.sync_copy(
            lists_hbm.at[pl.ds(
                pl.multiple_of(wid * LISTCAP + base_sc, 8), SCK)],
            lbuf)

        def unpack_body(g, carry):
            p = lbuf[pl.ds(g * 16, 16)]
            valid = (base_sc + g * 16 + iota) < cnt_splat
            lbuf[pl.ds(g * 16, 16)] = jnp.where(valid, p & PACK_MASK, 0)
            gdl[pl.ds(g * 16, 16)] = jnp.where(
                valid, lax.shift_right_logical(p, PACK_SHIFT),
                jnp.int32(DUMP))
            return carry
        lax.fori_loop(0, SCK // 16, unpack_body, 0)

        nedge_sc = jnp.clip(cnt - base_sc, 0, SCK)
        nblk = (nedge_sc + (GB - 1)) // GB

        @pl.when(nblk > 0)
        def _():
            gather_blk(0, rows0, sem0)

        def pair_body(pp, carry):
            e = 2 * pp
            o = e + 1

            @pl.when(o < nblk)
            def _():
                gather_blk(o, rows1, sem1)

            pltpu.make_async_copy(
                a_hbm.at[lbuf.at[pl.ds(pl.multiple_of(e * GB, 8), GB)]],
                rows0, sem0).wait()
            process_blk(e, rows0)

            @pl.when(e + 2 < nblk)
            def _():
                gather_blk(e + 2, rows0, sem0)

            @pl.when(o < nblk)
            def _():
                pltpu.make_async_copy(
                    a_hbm.at[lbuf.at[pl.ds(pl.multiple_of(o * GB, 8), GB)]],
                    rows1, sem1).wait()
                process_blk(o, rows1)
            return carry
        lax.fori_loop(0, (nblk + 1) // 2, pair_body, 0)
        return carry

    lax.fori_loop(0, (cnt + (SCK - 1)) // SCK, sc_body, 0)

    pltpu.sync_copy(acc.at[pl.ds(0, NODES_PER_TILE)],
                    s_hbm.at[pl.ds(lo, NODES_PER_TILE)])


def _dot(a, b):
    return jnp.dot(a, b, preferred_element_type=jnp.float32)


def _enc_body(f_ref, w1_ref, b1_ref, w2_ref, b2_ref, tw_ref, pw_ref,
              tb_ref, pb_ref, a_ref, bout_ref):
    x = jnp.maximum(_dot(f_ref[...], w1_ref[...]) + b1_ref[...], 0.0)
    x = jnp.maximum(_dot(x, w2_ref[...]) + b2_ref[...], 0.0)
    a_ref[...] = _dot(x, tw_ref[...])
    bout_ref[...] = (_dot(x, pw_ref[...] - tw_ref[...])
                     + (tb_ref[...] + pb_ref[...]))


def _layer_body(s_ref, bmat_ref, tw_ref, pw_ref, tb_ref, pb_ref,
                a_ref, bout_ref):
    x = jnp.maximum(bmat_ref[...] + s_ref[...], 0.0)
    a_ref[...] = _dot(x, tw_ref[...])
    bout_ref[...] = (_dot(x, pw_ref[...] - tw_ref[...])
                     + (tb_ref[...] + pb_ref[...]))


def _dec_body(s_ref, bmat_ref, w1_ref, b1_ref, w2_ref, b2_ref, out_ref):
    x = jnp.maximum(bmat_ref[...] + s_ref[...], 0.0)
    h = jnp.maximum(_dot(x, w1_ref[...]) + b1_ref[...], 0.0)
    o = _dot(h, w2_ref[...]) + b2_ref[...]
    out_ref[...] = jnp.where(o < THRESHOLD, 0.0, o)


_row_spec = pl.BlockSpec((ROW_BLK, H), lambda i: (i, 0))
_w_spec = pl.BlockSpec((H, H), lambda i: (0, 0))
_b_spec = pl.BlockSpec((1, H), lambda i: (0, 0))
_out2 = (jax.ShapeDtypeStruct((NP, H), jnp.float32),
         jax.ShapeDtypeStruct((NP, H), jnp.float32))

_enc_call = pl.pallas_call(
    _enc_body,
    grid=(NP // ROW_BLK,),
    in_specs=[_row_spec, _w_spec, _b_spec, _w_spec, _b_spec,
              _w_spec, _w_spec, _b_spec, _b_spec],
    out_specs=(_row_spec, _row_spec),
    out_shape=_out2,
)

_layer_call = pl.pallas_call(
    _layer_body,
    grid=(NP // ROW_BLK,),
    in_specs=[_row_spec, _row_spec, _w_spec, _w_spec, _b_spec, _b_spec],
    out_specs=(_row_spec, _row_spec),
    out_shape=_out2,
)

_dec_call = pl.pallas_call(
    _dec_body,
    grid=(NP // ROW_BLK,),
    in_specs=[_row_spec, _row_spec, _w_spec, _b_spec, _w_spec, _b_spec],
    out_specs=_row_spec,
    out_shape=jax.ShapeDtypeStruct((NP, H), jnp.float32),
)


def kernel(feats, edge_index, enc1_W, enc1_b, enc2_W, enc2_b,
           theta0_W, theta0_b, phi0_W, phi0_b,
           theta1_W, theta1_b, phi1_W, phi1_b,
           theta2_W, theta2_b, phi2_W, phi2_b,
           dec1_W, dec1_b, dec2_W, dec2_b):
    src = edge_index[0]
    dst = edge_index[1]
    featsp = jnp.pad(feats, ((0, NP - N), (0, 0)))
    r = lambda v: v.reshape(1, H)

    lists, counts = _build_lists(src, dst)

    A, B = _enc_call(featsp, enc1_W, r(enc1_b), enc2_W, r(enc2_b),
                     theta0_W, phi0_W, r(theta0_b), r(phi0_b))
    S = _segmax(lists, counts, A)
    A, B = _layer_call(S, B, theta1_W, phi1_W, r(theta1_b), r(phi1_b))
    S = _segmax(lists, counts, A)
    A, B = _layer_call(S, B, theta2_W, phi2_W, r(theta2_b), r(phi2_b))
    S = _segmax(lists, counts, A)

    dec2_Wp = jnp.pad(dec2_W, ((0, 0), (0, H - dec2_W.shape[1])))
    dec2_bp = jnp.pad(dec2_b, (0, H - dec2_b.shape[0]))
    out = _dec_call(S, B, dec1_W, r(dec1_b), dec2_Wp, dec2_bp.reshape(1, H))
    return out[:N, :1]


# speedup vs baseline: 4.1639x; 1.1013x over previous
import functools

import jax
import jax.numpy as jnp
from jax import lax
from jax.experimental import pallas as pl
from jax.experimental.pallas import tpu as pltpu
from jax.experimental.pallas import tpu_sc as plsc

N = 10000
E = 320000
H = 128
THRESHOLD = 0.01

NUM_WORKERS = 32
NODES_PER_TILE = 320
NP = NUM_WORKERS * NODES_PER_TILE

CHUNK = 8192
GROUPS = CHUNK // 16
LISTCAP = E + CHUNK
GB = 128
ROW_BLK = 512

PACK_SHIFT = 14
PACK_MASK = (1 << PACK_SHIFT) - 1

_mesh = plsc.VectorSubcoreMesh(
    core_axis_name="c", subcore_axis_name="s", num_cores=2, num_subcores=16)
_sc_params = pltpu.CompilerParams(needs_layout_passes=False)


def _wid():
    return lax.axis_index("s") * 2 + lax.axis_index("c")


@functools.partial(
    pl.kernel,
    out_type=(jax.ShapeDtypeStruct((NUM_WORKERS * LISTCAP,), jnp.int32),
              jax.ShapeDtypeStruct((NUM_WORKERS * 16,), jnp.int32)),
    mesh=_mesh,
    compiler_params=_sc_params,
    scratch_types=[
        pltpu.VMEM((CHUNK,), jnp.int32),
        pltpu.VMEM((CHUNK,), jnp.int32),
        pltpu.VMEM((2 * CHUNK,), jnp.int32),
        pltpu.VMEM((16,), jnp.int32),
    ],
)
def _build_lists(src_hbm, dst_hbm, lists_hbm, counts_hbm,
                 dstv, srcv, buf, cntv):
    wid = _wid()
    lo = wid * NODES_PER_TILE

    def chunk_body(k, carry):
        cnt_v, fl = carry
        pltpu.sync_copy(dst_hbm.at[pl.ds(pl.multiple_of(k * CHUNK, 8), CHUNK)], dstv)
        pltpu.sync_copy(src_hbm.at[pl.ds(pl.multiple_of(k * CHUNK, 8), CHUNK)], srcv)

        def f_body(g, cnt_v):
            d = dstv[pl.ds(g * 16, 16)]
            s = srcv[pl.ds(g * 16, 16)]
            m = (d >= lo) & (d < lo + NODES_PER_TILE)
            c = plsc.cumsum(jnp.where(m, jnp.int32(1), jnp.int32(0)))
            pos = (cnt_v - 1) + c
            packed = ((d - lo) << PACK_SHIFT) | s
            plsc.store_scatter(buf, [pos], packed, mask=m)
            return cnt_v + plsc.all_reduce_population_count(m)
        cnt_v = lax.fori_loop(0, GROUPS, f_body, cnt_v,
                              unroll=4)

        cb = cnt_v[0]

        @pl.when(cb >= CHUNK)
        def _flush():
            pltpu.sync_copy(buf.at[pl.ds(0, CHUNK)],
                            lists_hbm.at[pl.ds(pl.multiple_of(wid * LISTCAP + fl, 8), CHUNK)])
            nmv = (cb - CHUNK + 15) // 16

            def mv(k2, carry):
                v = buf[pl.ds(CHUNK + k2 * 16, 16)]
                buf[pl.ds(k2 * 16, 16)] = v
                return carry
            lax.fori_loop(0, nmv, mv, 0)

        did = jnp.where(cb >= CHUNK, jnp.int32(CHUNK), jnp.int32(0))
        return cnt_v - did, fl + did

    cnt_v0 = jnp.zeros((16,), jnp.int32)
    cnt_v, fl = lax.fori_loop(0, E // CHUNK, chunk_body,
                              (cnt_v0, jnp.int32(0)))
    pltpu.sync_copy(buf.at[pl.ds(0, CHUNK)],
                    lists_hbm.at[pl.ds(pl.multiple_of(wid * LISTCAP + fl, 8), CHUNK)])
    cntv[pl.ds(0, 16)] = cnt_v + fl
    pltpu.sync_copy(cntv, counts_hbm.at[pl.ds(pl.multiple_of(wid * 16, 8), 16)])


SCK = 4096
DUMP = NODES_PER_TILE


@functools.partial(
    pl.kernel,
    out_type=jax.ShapeDtypeStruct((NP, H), jnp.float32),
    mesh=_mesh,
    compiler_params=_sc_params,
    scratch_types=[
        [pltpu.VMEM(((NODES_PER_TILE + 8) * 16,), jnp.float32)
         for _ in range(8)],
        pltpu.VMEM((SCK,), jnp.int32),
        pltpu.VMEM((SCK,), jnp.int32),
        pltpu.VMEM((GB, H), jnp.float32),
        pltpu.VMEM((GB, H), jnp.float32),
        pltpu.VMEM((16,), jnp.int32),
        pltpu.SemaphoreType.DMA,
        pltpu.SemaphoreType.DMA,
    ],
)
def _segmax(lists_hbm, counts_hbm, a_hbm, s_hbm,
            accs, lbuf, gdl, rows0, rows1, cntv, sem0, sem1):
    wid = _wid()
    lo = wid * NODES_PER_TILE

    neg = jnp.full((16,), -jnp.inf, jnp.float32)

    def init_body(r, carry):
        for j in range(8):
            accs[j][pl.ds(pl.multiple_of(r * 16, 8), 16)] = neg
        return carry
    lax.fori_loop(0, NODES_PER_TILE + 8, init_body, 0)

    pltpu.sync_copy(counts_hbm.at[pl.ds(pl.multiple_of(wid * 16, 8), 16)],
                    cntv)
    cnt = cntv[pl.ds(0, 16)][0]
    iota = lax.iota(jnp.int32, 16)
    cnt_splat = cnt + jnp.zeros((16,), jnp.int32)
    iotas = [iota + 16 * j for j in range(8)]

    def gather_blk(b, rbuf, sem):
        return pltpu.async_copy(
            a_hbm.at[lbuf.at[pl.ds(pl.multiple_of(b * GB, 8), GB)]],
            rbuf, sem)

    def process_blk(b, rbuf):
        def grp(g2, carry):
            off = pl.multiple_of(b * GB + g2 * 16, 8)
            dlv = gdl[pl.ds(off, 16)]
            for i2 in range(16):
                dlb = dlv.at[jnp.full((16,), i2, jnp.int32)].get(
                    mode="promise_in_bounds")
                addr = dlb * 16 + iota
                for j in range(8):
                    ag = plsc.load_gather(accs[j], [addr])
                    rv = rbuf[g2 * 16 + i2, pl.ds(16 * j, 16)]
                    plsc.store_scatter(accs[j], [addr],
                                       jnp.maximum(ag, rv))
            return carry
        lax.fori_loop(0, GB // 16, grp, 0)

    def sc_body(scix, carry):
        base_sc = scix * SCK
        pltpu.sync_copy(
            lists_hbm.at[pl.ds(
                pl.multiple_of(wid * LISTCAP + base_sc, 8), SCK)],
            lbuf)

        def unpack_body(g, carry):
            p = lbuf[pl.ds(g * 16, 16)]
            valid = (base_sc + g * 16 + iota) < cnt_splat
            lbuf[pl.ds(g * 16, 16)] = jnp.where(valid, p & PACK_MASK, 0)
            gdl[pl.ds(g * 16, 16)] = jnp.where(
                valid, lax.shift_right_logical(p, PACK_SHIFT),
                jnp.int32(DUMP))
            return carry
        lax.fori_loop(0, SCK // 16, unpack_body, 0, unroll=4)

        nedge_sc = jnp.clip(cnt - base_sc, 0, SCK)
        nblk = (nedge_sc + (GB - 1)) // GB

        @pl.when(nblk > 0)
        def _():
            gather_blk(0, rows0, sem0)

        def pair_body(pp, carry):
            e = 2 * pp
            o = e + 1

            @pl.when(o < nblk)
            def _():
                gather_blk(o, rows1, sem1)

            pltpu.make_async_copy(
                a_hbm.at[lbuf.at[pl.ds(pl.multiple_of(e * GB, 8), GB)]],
                rows0, sem0).wait()
            process_blk(e, rows0)

            @pl.when(e + 2 < nblk)
            def _():
                gather_blk(e + 2, rows0, sem0)

            @pl.when(o < nblk)
            def _():
                pltpu.make_async_copy(
                    a_hbm.at[lbuf.at[pl.ds(pl.multiple_of(o * GB, 8), GB)]],
                    rows1, sem1).wait()
                process_blk(o, rows1)
            return carry
        lax.fori_loop(0, (nblk + 1) // 2, pair_body, 0)
        return carry

    lax.fori_loop(0, (cnt + (SCK - 1)) // SCK, sc_body, 0)

    for r0, nr in ((0, 128), (128, 128), (256, 64)):
        def wb(rr, carry):
            for j in range(8):
                rows0[rr, pl.ds(16 * j, 16)] = accs[j][
                    pl.ds(pl.multiple_of((r0 + rr) * 16, 8), 16)]
            return carry
        lax.fori_loop(0, nr, wb, 0)
        pltpu.sync_copy(rows0.at[pl.ds(0, nr)],
                        s_hbm.at[pl.ds(lo + r0, nr)])


def _dot(a, b):
    return jnp.dot(a, b, preferred_element_type=jnp.float32)


def _enc_body(f_ref, w1_ref, b1_ref, w2_ref, b2_ref, tw_ref, pw_ref,
              tb_ref, pb_ref, a_ref, bout_ref):
    x = jnp.maximum(_dot(f_ref[...], w1_ref[...]) + b1_ref[...], 0.0)
    x = jnp.maximum(_dot(x, w2_ref[...]) + b2_ref[...], 0.0)
    a_ref[...] = _dot(x, tw_ref[...])
    bout_ref[...] = (_dot(x, pw_ref[...] - tw_ref[...])
                     + (tb_ref[...] + pb_ref[...]))


def _layer_body(s_ref, bmat_ref, tw_ref, pw_ref, tb_ref, pb_ref,
                a_ref, bout_ref):
    x = jnp.maximum(bmat_ref[...] + s_ref[...], 0.0)
    a_ref[...] = _dot(x, tw_ref[...])
    bout_ref[...] = (_dot(x, pw_ref[...] - tw_ref[...])
                     + (tb_ref[...] + pb_ref[...]))


def _dec_body(s_ref, bmat_ref, w1_ref, b1_ref, w2_ref, b2_ref, out_ref):
    x = jnp.maximum(bmat_ref[...] + s_ref[...], 0.0)
    h = jnp.maximum(_dot(x, w1_ref[...]) + b1_ref[...], 0.0)
    o = _dot(h, w2_ref[...]) + b2_ref[...]
    out_ref[...] = jnp.where(o < THRESHOLD, 0.0, o)


_row_spec = pl.BlockSpec((ROW_BLK, H), lambda i: (i, 0))
_w_spec = pl.BlockSpec((H, H), lambda i: (0, 0))
_b_spec = pl.BlockSpec((1, H), lambda i: (0, 0))
_out2 = (jax.ShapeDtypeStruct((NP, H), jnp.float32),
         jax.ShapeDtypeStruct((NP, H), jnp.float32))

_enc_call = pl.pallas_call(
    _enc_body,
    grid=(NP // ROW_BLK,),
    in_specs=[_row_spec, _w_spec, _b_spec, _w_spec, _b_spec,
              _w_spec, _w_spec, _b_spec, _b_spec],
    out_specs=(_row_spec, _row_spec),
    out_shape=_out2,
)

_layer_call = pl.pallas_call(
    _layer_body,
    grid=(NP // ROW_BLK,),
    in_specs=[_row_spec, _row_spec, _w_spec, _w_spec, _b_spec, _b_spec],
    out_specs=(_row_spec, _row_spec),
    out_shape=_out2,
)

_dec_call = pl.pallas_call(
    _dec_body,
    grid=(NP // ROW_BLK,),
    in_specs=[_row_spec, _row_spec, _w_spec, _b_spec, _w_spec, _b_spec],
    out_specs=_row_spec,
    out_shape=jax.ShapeDtypeStruct((NP, H), jnp.float32),
)


def kernel(feats, edge_index, enc1_W, enc1_b, enc2_W, enc2_b,
           theta0_W, theta0_b, phi0_W, phi0_b,
           theta1_W, theta1_b, phi1_W, phi1_b,
           theta2_W, theta2_b, phi2_W, phi2_b,
           dec1_W, dec1_b, dec2_W, dec2_b):
    src = edge_index[0]
    dst = edge_index[1]
    featsp = jnp.pad(feats, ((0, NP - N), (0, 0)))
    r = lambda v: v.reshape(1, H)

    lists, counts = _build_lists(src, dst)

    A, B = _enc_call(featsp, enc1_W, r(enc1_b), enc2_W, r(enc2_b),
                     theta0_W, phi0_W, r(theta0_b), r(phi0_b))
    S = _segmax(lists, counts, A)
    A, B = _layer_call(S, B, theta1_W, phi1_W, r(theta1_b), r(phi1_b))
    S = _segmax(lists, counts, A)
    A, B = _layer_call(S, B, theta2_W, phi2_W, r(theta2_b), r(phi2_b))
    S = _segmax(lists, counts, A)

    dec2_Wp = jnp.pad(dec2_W, ((0, 0), (0, H - dec2_W.shape[1])))
    dec2_bp = jnp.pad(dec2_b, (0, H - dec2_b.shape[0]))
    out = _dec_call(S, B, dec1_W, r(dec1_b), dec2_Wp, dec2_bp.reshape(1, H))
    return out[:N, :1]
